# baseline (speedup 1.0000x reference)
import functools

import jax
import jax.numpy as jnp
from jax import lax
from jax.experimental import pallas as pl
from jax.experimental.pallas import tpu as pltpu
from jax.experimental.pallas import tpu_sc as plsc

_B = 4
_G = 200
_G2 = 256
_N = 121
_C = 3
_P = 224 * 224
_PB = 3584
_NB = _P // _PB
_K = 8


def _sc_gather(table, idx, total_rows, rows_per_worker):
    info = plsc.get_sparse_core_info()
    nc, ns = info.num_cores, info.num_subcores
    mesh = plsc.VectorSubcoreMesh(core_axis_name="c", subcore_axis_name="s")

    @functools.partial(
        pl.kernel,
        mesh=mesh,
        out_type=jax.ShapeDtypeStruct((total_rows, 128), jnp.float32),
        scratch_types=[
            pltpu.VMEM((rows_per_worker,), jnp.int32),
            pltpu.VMEM((rows_per_worker, 128), jnp.float32),
            pltpu.SemaphoreType.DMA,
        ],
    )
    def k(table_hbm, idx_hbm, out_hbm, idx_v, rows_v, sem):
        wid = lax.axis_index("s") * nc + lax.axis_index("c")
        base = wid * rows_per_worker
        pltpu.sync_copy(idx_hbm.at[pl.ds(base, rows_per_worker)], idx_v)
        pltpu.async_copy(table_hbm.at[idx_v], rows_v, sem).wait()
        pltpu.sync_copy(rows_v, out_hbm.at[pl.ds(base, rows_per_worker)])

    return k(table, idx)


def _gather_lat(t8_ref, gp_ref, lat_scr):
    gp = gp_ref[0]
    gph = gp.astype(jnp.bfloat16)
    r1 = gp - gph.astype(jnp.float32)
    gpm = r1.astype(jnp.bfloat16)
    gpl = (r1 - gpm.astype(jnp.float32)).astype(jnp.bfloat16)
    bmat = jnp.concatenate(
        [gph, gpm, gpl, jnp.zeros((128, 2), jnp.bfloat16)], axis=1)
    n_iota = lax.broadcasted_iota(jnp.int32, (1, 128), 1)
    for v in range(3):
        tv = t8_ref[0][:, v:v + 1]
        oh = (tv == n_iota).astype(jnp.bfloat16)
        cv = lax.dot_general(
            oh, bmat, (((1,), (0,)), ((), ())),
            preferred_element_type=jnp.float32)
        lat_scr[:, 2 * v:2 * v + 1] = cv[:, 0:1] + cv[:, 2:3] + cv[:, 4:5]
        lat_scr[:, 2 * v + 1:2 * v + 2] = cv[:, 1:2] + cv[:, 3:4] + cv[:, 5:6]


def _body(scal_ref, t8_ref, gp_ref, am_ref, pos_ref, fea_ref,
          cond_ref, recon_ref, stats_ref, av_ref, lat_scr, seg_scr):
    i = pl.program_id(1)

    @pl.when(i == 0)
    def _():
        _gather_lat(t8_ref, gp_ref, lat_scr)

    lat = lat_scr[...]
    x0, y0 = lat[:, 0:1], lat[:, 1:2]
    x1, y1 = lat[:, 2:3], lat[:, 3:4]
    x2, y2 = lat[:, 4:5], lat[:, 5:6]
    qx = pos_ref[0:1, :]
    qy = pos_ref[1:2, :]
    fea = fea_ref[0]
    s_iota = lax.broadcasted_iota(jnp.int32, (_G2, 1), 0)

    @pl.when(i < _NB)
    def _():
        xmin = jnp.minimum(jnp.minimum(x0, x1), x2)
        xmax = jnp.maximum(jnp.maximum(x0, x1), x2)
        ymin = jnp.minimum(jnp.minimum(y0, y1), y2)
        ymax = jnp.maximum(jnp.maximum(y0, y1), y2)
        inside = (qx >= xmin) & (qx <= xmax) & (qy >= ymin) & (qy <= ymax)
        g_f = s_iota.astype(jnp.float32)
        candf = jnp.where(inside, g_f, float(_G2))
        condf = jnp.min(candf, axis=0, keepdims=True)
        condf = jnp.where(condf == float(_G2), 0.0, condf)
        cond_ref[0, 0] = condf.astype(jnp.int32)

        onehot = (g_f == condf).astype(jnp.bfloat16)
        f_hi = fea.astype(jnp.bfloat16)
        f_lo = (fea - f_hi.astype(jnp.float32)).astype(jnp.bfloat16)
        vals16 = jnp.concatenate(
            [f_hi, jnp.ones((_PB, 1), jnp.bfloat16),
             jnp.zeros((_PB, 4), jnp.bfloat16),
             f_lo, jnp.zeros((_PB, 5), jnp.bfloat16)], axis=1)
        c16 = lax.dot_general(
            onehot, vals16, (((1,), (0,)), ((), ())),
            preferred_element_type=jnp.float32,
        )
        contrib = jnp.concatenate(
            [c16[:, 0:4] + c16[:, 8:12], jnp.zeros((_G2, 4), jnp.float32)],
            axis=1)

        @pl.when(i == 0)
        def _():
            seg_scr[...] = contrib

        @pl.when(i > 0)
        def _():
            seg_scr[...] += contrib

    @pl.when(i >= _NB)
    def _():
        neg_inv_sigma = scal_ref[0]
        cx = (x0 + x1 + x2) / 3.0
        cy = (y0 + y1 + y2) / 3.0
        cx = jnp.where(s_iota < _G, cx, 1e9)

        seg = seg_scr[...]
        cnt = jnp.maximum(seg[:, 3:4], 1.0)
        gf = seg[:, 0:3] / cnt
        g2m = jnp.sum(gf * gf, axis=1, keepdims=True) * (1.0 / _C)
        gfe = jnp.concatenate(
            [gf, g2m, jnp.ones((_G2, 1), jnp.float32),
             jnp.zeros((_G2, 3), jnp.float32)], axis=1)

        dx = qx - cx
        dy = qy - cy
        d2 = dx * dx + dy * dy
        logits = jnp.minimum(d2 * neg_inv_sigma, -1e-30)

        lb = lax.bitcast_convert_type(logits, jnp.int32)
        keyi = (~lb & jnp.int32(-256)) | (jnp.int32(255) - s_iota)
        key = lax.bitcast_convert_type(keyi, jnp.float32)
        m0k = jnp.max(key, axis=0, keepdims=True)
        for k in range(_K):
            mk = m0k if k == 0 else jnp.max(key, axis=0, keepdims=True)
            key = jnp.where(key == mk, 0.0, key)
        selm = key == 0.0
        lprime = lax.bitcast_convert_type(lb & jnp.int32(-256), jnp.float32)
        m0i = lax.bitcast_convert_type(m0k, jnp.int32)
        m0p = lax.bitcast_convert_type(~m0i & jnp.int32(-256), jnp.float32)
        wb = jnp.where(selm, jnp.exp(lprime - m0p), 0.0).astype(jnp.bfloat16)

        g_hi = gfe.astype(jnp.bfloat16)
        g_lo = (gfe - g_hi.astype(jnp.float32)).astype(jnp.bfloat16)
        gpack = jnp.concatenate([g_hi, g_lo], axis=1)
        r16 = lax.dot_general(
            wb, gpack, (((0,), (0,)), ((), ())),
            preferred_element_type=jnp.float32,
        )
        r4 = r16[:, 0:8] + r16[:, 8:16]
        r4 = r4 / r4[:, 4:5]
        recon = r4[:, 0:3]
        recon_ref[0, 0] = recon

        fea2m = jnp.sum(fea * fea, axis=1, keepdims=True) * (1.0 / _C)
        dotfr = jnp.sum(fea * recon, axis=1, keepdims=True)
        varp = fea2m - (2.0 / _C) * dotfr + r4[:, 3:4]
        lossp = jnp.sum(jnp.abs(recon - fea), axis=1, keepdims=True)
        part = jnp.concatenate(
            [varp, lossp, jnp.zeros((_PB, 6), jnp.float32)], axis=1)

        @pl.when(i == _NB)
        def _():
            stats_ref[0] = part

        @pl.when(i > _NB)
        def _():
            stats_ref[0] += part

        @pl.when(i == _NB)
        def _():
            am = am_ref[0]
            ax, ay = 20.0 * x0, 20.0 * y0
            bx, by = 20.0 * x1, 20.0 * y1
            cx2, cy2 = 20.0 * x2, 20.0 * y2
            area1 = (ay + by) * (bx - ax) * 0.5
            area2 = (by + cy2) * (cx2 - bx) * 0.5
            area3 = (cy2 + ay) * (ax - cx2) * 0.5
            area = (area1 + area2 + area3) * am
            lane_valid = s_iota < _G
            area = jnp.where(lane_valid, area, 0.0)
            mean = jnp.sum(area) / _G
            dev = jnp.where(lane_valid, area - mean, 0.0)
            av_ref[...] = jnp.broadcast_to(
                jnp.sum(dev * dev) / (_G - 1), (1, 1, 1))


def kernel(grid_pos, img_fea, base_triangle2point, base_area_mask,
           base_triangle_mask, grid_size, output_pos):
    del base_triangle_mask
    B, N = grid_pos.shape[0], grid_pos.shape[1]
    G = base_triangle2point.shape[1]

    table = jnp.pad(grid_pos.reshape(B * N, 2), ((0, 0), (0, 126)))
    idx = (base_triangle2point.reshape(B, G * 3)
           + (jnp.arange(B, dtype=jnp.int32) * N)[:, None]).reshape(-1)
    total = B * G * 3
    total_pad = 2560
    idx = jnp.pad(idx, (0, total_pad - total))
    rows = _sc_gather(table, idx, total_pad, total_pad // 32)
    lattice = rows[:total, :2].reshape(B, G, 3, 2)

    t8 = jnp.pad(base_triangle2point, ((0, 0), (0, _G2 - G), (0, 1)),
                 constant_values=127)
    gp2 = jnp.pad(grid_pos, ((0, 0), (0, 128 - N), (0, 0)))
    am_col = jnp.pad(base_area_mask, ((0, 0), (0, _G2 - G)))[:, :, None]

    pos_t = output_pos.reshape(_P, 2).T
    fea = img_fea.reshape(B, _P, _C)

    max_grid = jnp.maximum(grid_size[0] - 1, grid_size[1] - 1).astype(jnp.float32)
    neg_inv_sigma = jnp.reshape(-max_grid / 0.02, (1,))

    cond4, recon4, stats, av = pl.pallas_call(
        _body,
        grid=(B, 2 * _NB),
        in_specs=[
            pl.BlockSpec(memory_space=pltpu.SMEM),
            pl.BlockSpec((1, _G2, 4), lambda b, i: (b, 0, 0)),
            pl.BlockSpec((1, 128, 2), lambda b, i: (b, 0, 0)),
            pl.BlockSpec((1, _G2, 1), lambda b, i: (b, 0, 0)),
            pl.BlockSpec((2, _PB), lambda b, i: (0, i % _NB)),
            pl.BlockSpec((1, _PB, _C), lambda b, i: (b, i % _NB, 0)),
        ],
        out_specs=[
            pl.BlockSpec((1, 1, 1, _PB),
                         lambda b, i: (b, jnp.minimum(i, _NB), 0, 0)),
            pl.BlockSpec((1, 1, _PB, _C),
                         lambda b, i: (b, jnp.where(i < _NB, _NB, i - _NB),
                                       0, 0)),
            pl.BlockSpec((1, _PB, 8), lambda b, i: (b, 0, 0)),
            pl.BlockSpec((1, 1, 1), lambda b, i: (b, 0, 0)),
        ],
        out_shape=[
            jax.ShapeDtypeStruct((B, _NB + 1, 1, _PB), jnp.int32),
            jax.ShapeDtypeStruct((B, _NB + 1, _PB, _C), jnp.float32),
            jax.ShapeDtypeStruct((B, _PB, 8), jnp.float32),
            jax.ShapeDtypeStruct((B, 1, 1), jnp.float32),
        ],
        scratch_shapes=[
            pltpu.VMEM((_G2, 8), jnp.float32),
            pltpu.VMEM((_G2, 8), jnp.float32),
        ],
        compiler_params=pltpu.CompilerParams(
            dimension_semantics=("arbitrary", "arbitrary"),
        ),
    )(neg_inv_sigma, t8, gp2, am_col, pos_t, fea)

    condition = cond4[:, :_NB].reshape(B, _P, 1)
    variance = jnp.sum(stats[:, :, 0], axis=1) / _P
    reconstruct_loss = jnp.sum(stats[:, :, 1], axis=1) / (_P * _C)
    area_variance = av[:, 0, 0]
    recon_img = recon4[:, :_NB].reshape(B, 224, 224, _C)
    return (condition, lattice, variance, area_variance,
            reconstruct_loss, recon_img)
